# baseline (speedup 1.0000x reference)
import functools

import jax
import jax.numpy as jnp
from jax import lax
from jax.experimental import pallas as pl
from jax.experimental.pallas import tpu as pltpu
from jax.experimental.pallas import tpu_sc as plsc

_THRESHOLDS = (0.25, 0.375, 0.5, 0.625)
_NCLASS = 5
_SMOOTH = 1.0

_B, _D, _C, _H, _W = 2, 16, 5, 256, 256
_P = _H * _W
_NW = 32
_LANES = 16
_CROWS = 32
_CH = _CROWS * _W
_NCHUNK = _P // _CH
_STEPS = _CH // _LANES
_NACC = 14
_ACC_ROWS = 16
_UNROLL = 8
_KSC = 2
_KTC = _NCHUNK - _KSC
_RT = _CROWS * _KTC
_DBLK = 8


def _dice_body(inp_hbm, tgt_hbm, out_hbm,
               inp_v0, tgt_v0, inp_v1, tgt_v1, stage_v, sem0, sem1):
    wid = lax.axis_index("s") * 2 + lax.axis_index("c")
    b = wid // _D
    d = wid % _D

    zero = jnp.zeros((_LANES,), jnp.float32)
    bufs = ((inp_v0, tgt_v0, sem0), (inp_v1, tgt_v1, sem1))
    pending = {}

    def start(k, p):
        inp_v, tgt_v, sem = bufs[p]
        h0 = (_KTC + k) * _CROWS
        ds = [pltpu.async_copy(tgt_hbm.at[b, d, 0, pl.ds(h0, _CROWS), :],
                               tgt_v, sem)]
        for c in range(_C):
            ds.append(pltpu.async_copy(
                inp_hbm.at[b, d, c, pl.ds(h0, _CROWS), :],
                inp_v.at[pl.ds(c * _CROWS, _CROWS), :], sem))
        pending[p] = ds

    def compute_chunk(inp_v, tgt_v, accs):
        @plsc.parallel_loop(0, _STEPS, unroll=_UNROLL, carry=accs)
        def body(idx, accs):
            (i0, i1, i2, i3, i4,
             s0, s1, s2, s3, s4, g0, g1, g2, g3) = accs
            r = idx // (_W // _LANES)
            col = (idx % (_W // _LANES)) * _LANES
            t = tgt_v[r, pl.ds(col, _LANES)]
            a0 = jnp.where(t >= _THRESHOLDS[0], 1.0, 0.0)
            a1 = jnp.where(t >= _THRESHOLDS[1], 1.0, 0.0)
            a2 = jnp.where(t >= _THRESHOLDS[2], 1.0, 0.0)
            a3 = jnp.where(t >= _THRESHOLDS[3], 1.0, 0.0)
            ci = a0 + a1 + a2 + a3
            xs = [inp_v[c * _CROWS + r, pl.ds(col, _LANES)]
                  for c in range(_C)]
            it = [jnp.where(ci == float(c), xs[c], 0.0)
                  for c in range(_C)]
            return (i0 + it[0], i1 + it[1], i2 + it[2], i3 + it[3],
                    i4 + it[4],
                    s0 + xs[0], s1 + xs[1], s2 + xs[2], s3 + xs[3],
                    s4 + xs[4],
                    g0 + a0, g1 + a1, g2 + a2, g3 + a3)

        return body

    start(0, 0)
    accs = (zero,) * _NACC
    for k in range(_KSC):
        p = k % 2
        if k + 1 < _KSC:
            start(k + 1, 1 - p)
        for dsc in pending[p]:
            dsc.wait()
        accs = compute_chunk(bufs[p][0], bufs[p][1], accs)

    for j in range(_NACC):
        stage_v[pl.ds(j * _LANES, _LANES)] = accs[j]
    for j in range(_NACC, _ACC_ROWS):
        stage_v[pl.ds(j * _LANES, _LANES)] = zero
    pltpu.sync_copy(stage_v, out_hbm.at[pl.ds(wid * _ACC_ROWS * _LANES,
                                              _ACC_ROWS * _LANES)])


def _tc_body(x_ref, t_ref, o_ref):
    accs = [None] * _NACC
    for g in range(_DBLK):
        t = t_ref[0, g, 0]
        a = [jnp.where(t >= th, 1.0, 0.0) for th in _THRESHOLDS]
        ci = a[0] + a[1] + a[2] + a[3]
        quants = ([jnp.where(ci == float(c), x_ref[0, g, c], 0.0)
                   for c in range(_C)]
                  + [x_ref[0, g, c] for c in range(_C)]
                  + a)
        for q, arr in enumerate(quants):
            acc = arr[0:8]
            for i in range(1, _RT // 8):
                acc = acc + arr[i * 8:(i + 1) * 8]
            acc = acc[:, :128] + acc[:, 128:]
            accs[q] = acc if accs[q] is None else accs[q] + acc
    for q in range(_NACC):
        o_ref[0, q] = accs[q]


@jax.jit
def kernel(input, target, weight):
    mesh = plsc.VectorSubcoreMesh(
        core_axis_name="c", subcore_axis_name="s", num_cores=2,
        num_subcores=16)
    partials = pl.kernel(
        _dice_body,
        out_type=jax.ShapeDtypeStruct((_NW * _ACC_ROWS * _LANES,),
                                      jnp.float32),
        mesh=mesh,
        scratch_types=[
            pltpu.VMEM((_C * _CROWS, _W), jnp.float32),
            pltpu.VMEM((_CROWS, _W), jnp.float32),
            pltpu.VMEM((_C * _CROWS, _W), jnp.float32),
            pltpu.VMEM((_CROWS, _W), jnp.float32),
            pltpu.VMEM((_ACC_ROWS * _LANES,), jnp.float32),
            pltpu.SemaphoreType.DMA,
            pltpu.SemaphoreType.DMA,
        ],
        compiler_params=pltpu.CompilerParams(use_tc_tiling_on_sc=True,
                                             skip_device_barrier=True),
    )(input, target)

    ngrid = _NW // _DBLK
    tc_part = pl.pallas_call(
        _tc_body,
        out_shape=jax.ShapeDtypeStruct((ngrid, _NACC, 8, 128),
                                       jnp.float32),
        grid=(ngrid,),
        in_specs=[
            pl.BlockSpec((1, _DBLK, _C, _RT, _W),
                         lambda p: (p // (_D // _DBLK), p % (_D // _DBLK),
                                    0, 0, 0)),
            pl.BlockSpec((1, _DBLK, 1, _RT, _W),
                         lambda p: (p // (_D // _DBLK), p % (_D // _DBLK),
                                    0, 0, 0)),
        ],
        out_specs=pl.BlockSpec((1, _NACC, 8, 128), lambda p: (p, 0, 0, 0)),
    )(input, target)

    p = partials.reshape(_B, _D, _ACC_ROWS, _LANES).sum(axis=(1, 3))
    p = p[:, :_NACC] + tc_part.sum(axis=(2, 3)).reshape(
        _B, ngrid // _B, _NACC).sum(axis=1)
    inter = p[:, 0:_NCLASS]
    sumi = p[:, _NCLASS:2 * _NCLASS]
    ge = p[:, 2 * _NCLASS:2 * _NCLASS + 4]
    total_px = jnp.float32(_D * _P)
    cnt = jnp.stack(
        [total_px - ge[:, 0]]
        + [ge[:, k - 1] - ge[:, k] for k in range(1, 4)]
        + [ge[:, 3]], axis=1)
    dice = 2.0 * (inter + _SMOOTH) / (sumi + cnt + _SMOOTH)
    loss_c = 1.0 - jnp.sum(dice, axis=0) / _B
    return jnp.sum(weight * loss_c)


# speedup vs baseline: 1.0020x; 1.0020x over previous
import functools

import jax
import jax.numpy as jnp
from jax import lax
from jax.experimental import pallas as pl
from jax.experimental.pallas import tpu as pltpu
from jax.experimental.pallas import tpu_sc as plsc

_THRESHOLDS = (0.25, 0.375, 0.5, 0.625)
_NCLASS = 5
_SMOOTH = 1.0

_B, _D, _C, _H, _W = 2, 16, 5, 256, 256
_P = _H * _W
_NW = 32
_LANES = 16
_CROWS = 32
_CH = _CROWS * _W
_NCHUNK = _P // _CH
_STEPS = _CH // _LANES
_NACC = 14
_ACC_ROWS = 16
_UNROLL = 8
_KSC = 2
_KTC = _NCHUNK - _KSC
_RT = _CROWS * _KTC
_DBLK = 4


def _dice_body(inp_hbm, tgt_hbm, out_hbm,
               inp_v0, tgt_v0, inp_v1, tgt_v1, stage_v, sem0, sem1):
    wid = lax.axis_index("s") * 2 + lax.axis_index("c")
    b = wid // _D
    d = wid % _D

    zero = jnp.zeros((_LANES,), jnp.float32)
    bufs = ((inp_v0, tgt_v0, sem0), (inp_v1, tgt_v1, sem1))
    pending = {}

    def start(k, p):
        inp_v, tgt_v, sem = bufs[p]
        h0 = (_KTC + k) * _CROWS
        ds = [pltpu.async_copy(tgt_hbm.at[b, d, 0, pl.ds(h0, _CROWS), :],
                               tgt_v, sem)]
        for c in range(_C):
            ds.append(pltpu.async_copy(
                inp_hbm.at[b, d, c, pl.ds(h0, _CROWS), :],
                inp_v.at[pl.ds(c * _CROWS, _CROWS), :], sem))
        pending[p] = ds

    def compute_chunk(inp_v, tgt_v, accs):
        @plsc.parallel_loop(0, _STEPS, unroll=_UNROLL, carry=accs)
        def body(idx, accs):
            (i0, i1, i2, i3, i4,
             s0, s1, s2, s3, s4, g0, g1, g2, g3) = accs
            r = idx // (_W // _LANES)
            col = (idx % (_W // _LANES)) * _LANES
            t = tgt_v[r, pl.ds(col, _LANES)]
            a0 = jnp.where(t >= _THRESHOLDS[0], 1.0, 0.0)
            a1 = jnp.where(t >= _THRESHOLDS[1], 1.0, 0.0)
            a2 = jnp.where(t >= _THRESHOLDS[2], 1.0, 0.0)
            a3 = jnp.where(t >= _THRESHOLDS[3], 1.0, 0.0)
            ci = a0 + a1 + a2 + a3
            xs = [inp_v[c * _CROWS + r, pl.ds(col, _LANES)]
                  for c in range(_C)]
            it = [jnp.where(ci == float(c), xs[c], 0.0)
                  for c in range(_C)]
            return (i0 + it[0], i1 + it[1], i2 + it[2], i3 + it[3],
                    i4 + it[4],
                    s0 + xs[0], s1 + xs[1], s2 + xs[2], s3 + xs[3],
                    s4 + xs[4],
                    g0 + a0, g1 + a1, g2 + a2, g3 + a3)

        return body

    start(0, 0)
    accs = (zero,) * _NACC
    for k in range(_KSC):
        p = k % 2
        if k + 1 < _KSC:
            start(k + 1, 1 - p)
        for dsc in pending[p]:
            dsc.wait()
        accs = compute_chunk(bufs[p][0], bufs[p][1], accs)

    for j in range(_NACC):
        stage_v[pl.ds(j * _LANES, _LANES)] = accs[j]
    for j in range(_NACC, _ACC_ROWS):
        stage_v[pl.ds(j * _LANES, _LANES)] = zero
    pltpu.sync_copy(stage_v, out_hbm.at[pl.ds(wid * _ACC_ROWS * _LANES,
                                              _ACC_ROWS * _LANES)])


def _tc_body(x_ref, t_ref, o_ref):
    accs = [None] * _NACC
    for g in range(_DBLK):
        t = t_ref[0, g, 0]
        a = [jnp.where(t >= th, 1.0, 0.0) for th in _THRESHOLDS]
        ci = a[0] + a[1] + a[2] + a[3]
        quants = ([jnp.where(ci == float(c), x_ref[0, g, c], 0.0)
                   for c in range(_C)]
                  + [x_ref[0, g, c] for c in range(_C)]
                  + a)
        for q, arr in enumerate(quants):
            acc = arr[0:8]
            for i in range(1, _RT // 8):
                acc = acc + arr[i * 8:(i + 1) * 8]
            acc = acc[:, :128] + acc[:, 128:]
            accs[q] = acc if accs[q] is None else accs[q] + acc
    for q in range(_NACC):
        o_ref[0, q] = accs[q]


@jax.jit
def kernel(input, target, weight):
    mesh = plsc.VectorSubcoreMesh(
        core_axis_name="c", subcore_axis_name="s", num_cores=2,
        num_subcores=16)
    partials = pl.kernel(
        _dice_body,
        out_type=jax.ShapeDtypeStruct((_NW * _ACC_ROWS * _LANES,),
                                      jnp.float32),
        mesh=mesh,
        scratch_types=[
            pltpu.VMEM((_C * _CROWS, _W), jnp.float32),
            pltpu.VMEM((_CROWS, _W), jnp.float32),
            pltpu.VMEM((_C * _CROWS, _W), jnp.float32),
            pltpu.VMEM((_CROWS, _W), jnp.float32),
            pltpu.VMEM((_ACC_ROWS * _LANES,), jnp.float32),
            pltpu.SemaphoreType.DMA,
            pltpu.SemaphoreType.DMA,
        ],
        compiler_params=pltpu.CompilerParams(use_tc_tiling_on_sc=True,
                                             skip_device_barrier=True),
    )(input, target)

    ngrid = _NW // _DBLK
    tc_part = pl.pallas_call(
        _tc_body,
        out_shape=jax.ShapeDtypeStruct((ngrid, _NACC, 8, 128),
                                       jnp.float32),
        grid=(ngrid,),
        in_specs=[
            pl.BlockSpec((1, _DBLK, _C, _RT, _W),
                         lambda p: (p // (_D // _DBLK), p % (_D // _DBLK),
                                    0, 0, 0)),
            pl.BlockSpec((1, _DBLK, 1, _RT, _W),
                         lambda p: (p // (_D // _DBLK), p % (_D // _DBLK),
                                    0, 0, 0)),
        ],
        out_specs=pl.BlockSpec((1, _NACC, 8, 128), lambda p: (p, 0, 0, 0)),
    )(input, target)

    p = partials.reshape(_B, _D, _ACC_ROWS, _LANES).sum(axis=(1, 3))
    p = p[:, :_NACC] + tc_part.sum(axis=(2, 3)).reshape(
        _B, ngrid // _B, _NACC).sum(axis=1)
    inter = p[:, 0:_NCLASS]
    sumi = p[:, _NCLASS:2 * _NCLASS]
    ge = p[:, 2 * _NCLASS:2 * _NCLASS + 4]
    total_px = jnp.float32(_D * _P)
    cnt = jnp.stack(
        [total_px - ge[:, 0]]
        + [ge[:, k - 1] - ge[:, k] for k in range(1, 4)]
        + [ge[:, 3]], axis=1)
    dice = 2.0 * (inter + _SMOOTH) / (sumi + cnt + _SMOOTH)
    loss_c = 1.0 - jnp.sum(dice, axis=0) / _B
    return jnp.sum(weight * loss_c)


# speedup vs baseline: 1.0508x; 1.0487x over previous
import functools

import jax
import jax.numpy as jnp
from jax import lax
from jax.experimental import pallas as pl
from jax.experimental.pallas import tpu as pltpu
from jax.experimental.pallas import tpu_sc as plsc

_THRESHOLDS = (0.25, 0.375, 0.5, 0.625)
_NCLASS = 5
_SMOOTH = 1.0

_B, _D, _C, _H, _W = 2, 16, 5, 256, 256
_P = _H * _W
_NW = 32
_LANES = 16
_CROWS = 32
_CH = _CROWS * _W
_NCHUNK = _P // _CH
_STEPS = _CH // _LANES
_NACC = 14
_ACC_ROWS = 16
_UNROLL = 8
_KSC = 2
_KTC = _NCHUNK - _KSC
_RT = _CROWS * _KTC
_DBLK = 4


def _dice_body(inp_hbm, tgt_hbm, out_hbm,
               inp_v0, tgt_v0, inp_v1, tgt_v1, stage_v, sem0, sem1):
    wid = lax.axis_index("s") * 2 + lax.axis_index("c")
    b = wid // _D
    d = wid % _D

    zero = jnp.zeros((_LANES,), jnp.float32)
    bufs = ((inp_v0, tgt_v0, sem0), (inp_v1, tgt_v1, sem1))
    pending = {}

    def start(k, p):
        inp_v, tgt_v, sem = bufs[p]
        h0 = (_KTC + k) * _CROWS
        ds = [pltpu.async_copy(tgt_hbm.at[b, d, 0, pl.ds(h0, _CROWS), :],
                               tgt_v, sem)]
        for c in range(_C):
            ds.append(pltpu.async_copy(
                inp_hbm.at[b, d, c, pl.ds(h0, _CROWS), :],
                inp_v.at[pl.ds(c * _CROWS, _CROWS), :], sem))
        pending[p] = ds

    def compute_chunk(inp_v, tgt_v, accs):
        @plsc.parallel_loop(0, _STEPS, unroll=_UNROLL, carry=accs)
        def body(idx, accs):
            (i0, i1, i2, i3, i4,
             s0, s1, s2, s3, s4, g0, g1, g2, g3) = accs
            r = idx // (_W // _LANES)
            col = (idx % (_W // _LANES)) * _LANES
            t = tgt_v[r, pl.ds(col, _LANES)]
            a0 = jnp.where(t >= _THRESHOLDS[0], 1.0, 0.0)
            a1 = jnp.where(t >= _THRESHOLDS[1], 1.0, 0.0)
            a2 = jnp.where(t >= _THRESHOLDS[2], 1.0, 0.0)
            a3 = jnp.where(t >= _THRESHOLDS[3], 1.0, 0.0)
            ci = a0 + a1 + a2 + a3
            xs = [inp_v[c * _CROWS + r, pl.ds(col, _LANES)]
                  for c in range(_C)]
            it = [jnp.where(ci == float(c), xs[c], 0.0)
                  for c in range(_C)]
            return (i0 + it[0], i1 + it[1], i2 + it[2], i3 + it[3],
                    i4 + it[4],
                    s0 + xs[0], s1 + xs[1], s2 + xs[2], s3 + xs[3],
                    s4 + xs[4],
                    g0 + a0, g1 + a1, g2 + a2, g3 + a3)

        return body

    start(0, 0)
    accs = (zero,) * _NACC
    for k in range(_KSC):
        p = k % 2
        if k + 1 < _KSC:
            start(k + 1, 1 - p)
        for dsc in pending[p]:
            dsc.wait()
        accs = compute_chunk(bufs[p][0], bufs[p][1], accs)

    for j in range(_NACC):
        stage_v[pl.ds(j * _LANES, _LANES)] = accs[j]
    for j in range(_NACC, _ACC_ROWS):
        stage_v[pl.ds(j * _LANES, _LANES)] = zero
    pltpu.sync_copy(stage_v, out_hbm.at[pl.ds(wid * _ACC_ROWS * _LANES,
                                              _ACC_ROWS * _LANES)])


def _tc_body(x_ref, t_ref, o_ref):
    accs = [None] * _NACC
    for g in range(_DBLK):
        t = t_ref[0, g, 0]
        a = [jnp.where(t >= th, 1.0, 0.0) for th in _THRESHOLDS]
        ci = a[0] + a[1] + a[2] + a[3]
        quants = ([jnp.where(ci == float(c), x_ref[0, g, c], 0.0)
                   for c in range(_C)]
                  + [x_ref[0, g, c] for c in range(_C)]
                  + a)
        for q, arr in enumerate(quants):
            acc = arr[0:8]
            for i in range(1, _RT // 8):
                acc = acc + arr[i * 8:(i + 1) * 8]
            acc = acc[:, :128] + acc[:, 128:]
            accs[q] = acc if accs[q] is None else accs[q] + acc
    for q in range(_NACC):
        o_ref[0, q] = accs[q]


@jax.jit
def kernel(input, target, weight):
    mesh = plsc.VectorSubcoreMesh(
        core_axis_name="c", subcore_axis_name="s", num_cores=2,
        num_subcores=16)
    partials = pl.kernel(
        _dice_body,
        out_type=jax.ShapeDtypeStruct((_NW * _ACC_ROWS * _LANES,),
                                      jnp.float32),
        mesh=mesh,
        scratch_types=[
            pltpu.VMEM((_C * _CROWS, _W), jnp.float32),
            pltpu.VMEM((_CROWS, _W), jnp.float32),
            pltpu.VMEM((_C * _CROWS, _W), jnp.float32),
            pltpu.VMEM((_CROWS, _W), jnp.float32),
            pltpu.VMEM((_ACC_ROWS * _LANES,), jnp.float32),
            pltpu.SemaphoreType.DMA,
            pltpu.SemaphoreType.DMA,
        ],
        compiler_params=pltpu.CompilerParams(use_tc_tiling_on_sc=True),
    )(input, target)

    ngrid = _NW // _DBLK
    tc_part = pl.pallas_call(
        _tc_body,
        out_shape=jax.ShapeDtypeStruct((ngrid, _NACC, 8, 128),
                                       jnp.float32),
        grid=(ngrid,),
        in_specs=[
            pl.BlockSpec((1, _DBLK, _C, _RT, _W),
                         lambda p: (p // (_D // _DBLK), p % (_D // _DBLK),
                                    0, 0, 0)),
            pl.BlockSpec((1, _DBLK, 1, _RT, _W),
                         lambda p: (p // (_D // _DBLK), p % (_D // _DBLK),
                                    0, 0, 0)),
        ],
        out_specs=pl.BlockSpec((1, _NACC, 8, 128), lambda p: (p, 0, 0, 0)),
    )(input, target)

    p = partials.reshape(_B, _D, _ACC_ROWS, _LANES).sum(axis=(1, 3))
    p = p[:, :_NACC] + tc_part.sum(axis=(2, 3)).reshape(
        _B, ngrid // _B, _NACC).sum(axis=1)
    inter = p[:, 0:_NCLASS]
    sumi = p[:, _NCLASS:2 * _NCLASS]
    ge = p[:, 2 * _NCLASS:2 * _NCLASS + 4]
    total_px = jnp.float32(_D * _P)
    cnt = jnp.stack(
        [total_px - ge[:, 0]]
        + [ge[:, k - 1] - ge[:, k] for k in range(1, 4)]
        + [ge[:, 3]], axis=1)
    dice = 2.0 * (inter + _SMOOTH) / (sumi + cnt + _SMOOTH)
    loss_c = 1.0 - jnp.sum(dice, axis=0) / _B
    return jnp.sum(weight * loss_c)
